# Initial kernel scaffold; baseline (speedup 1.0000x reference)
import functools

import jax
import jax.numpy as jnp
from jax import lax
from jax.experimental import pallas as pl
from jax.experimental.pallas import tpu as pltpu
from jax.experimental.pallas import tpu_sc as plsc

NUM_CORES = 2
NUM_SUBCORES = 16
NUM_WORKERS = NUM_CORES * NUM_SUBCORES
CHUNK = 64


def _make_gather(N, V, D):
    n_per_w = N // NUM_WORKERS
    n_chunks = n_per_w // CHUNK

    mesh = plsc.VectorSubcoreMesh(
        core_axis_name="c", subcore_axis_name="s",
        num_cores=NUM_CORES, num_subcores=NUM_SUBCORES)

    @functools.partial(
        pl.kernel,
        mesh=mesh,
        out_type=jax.ShapeDtypeStruct((N, D), jnp.float32),
        scratch_types=[
            pltpu.VMEM((n_per_w,), jnp.int32),
            pltpu.VMEM((CHUNK, D), jnp.float32),
            pltpu.SemaphoreType.DMA,
        ],
    )
    def gather_kernel(idx_hbm, table_hbm, out_hbm, idx_v, rows_v, gsem):
        wid = lax.axis_index("s") * NUM_CORES + lax.axis_index("c")
        base = wid * n_per_w
        pltpu.sync_copy(idx_hbm.at[pl.ds(base, n_per_w)], idx_v)

        def body(g, carry):
            off = g * CHUNK
            pltpu.async_copy(
                table_hbm.at[idx_v.at[pl.ds(off, CHUNK)]], rows_v, gsem
            ).wait()
            pltpu.sync_copy(rows_v, out_hbm.at[pl.ds(base + off, CHUNK)])
            return carry

        lax.fori_loop(0, n_chunks, body, 0)

    return gather_kernel


def kernel(idx, table):
    B, T = idx.shape
    V, D = table.shape
    N = B * T
    out = _make_gather(N, V, D)(idx.reshape(N).astype(jnp.int32), table)
    return out.reshape(B, T, D)


# speedup vs baseline: 1.0140x; 1.0140x over previous
import functools

import jax
import jax.numpy as jnp
from jax import lax
from jax.experimental import pallas as pl
from jax.experimental.pallas import tpu as pltpu
from jax.experimental.pallas import tpu_sc as plsc

NUM_CORES = 2
NUM_SUBCORES = 16
NUM_WORKERS = NUM_CORES * NUM_SUBCORES
CHUNK = 64


def _make_gather(N, V, D):
    n_per_w = N // NUM_WORKERS
    n_chunks = n_per_w // CHUNK

    mesh = plsc.VectorSubcoreMesh(
        core_axis_name="c", subcore_axis_name="s",
        num_cores=NUM_CORES, num_subcores=NUM_SUBCORES)

    @functools.partial(
        pl.kernel,
        mesh=mesh,
        compiler_params=pltpu.CompilerParams(use_tc_tiling_on_sc=False),
        out_type=jax.ShapeDtypeStruct((N, D), jnp.float32),
        scratch_types=[
            pltpu.VMEM((n_per_w,), jnp.int32),
            pltpu.VMEM((CHUNK, D), jnp.float32),
            pltpu.SemaphoreType.DMA,
        ],
    )
    def gather_kernel(idx_hbm, table_hbm, out_hbm, idx_v, rows_v, gsem):
        wid = lax.axis_index("s") * NUM_CORES + lax.axis_index("c")
        base = wid * n_per_w
        pltpu.sync_copy(idx_hbm.at[pl.ds(base, n_per_w)], idx_v)

        def body(g, carry):
            off = g * CHUNK
            pltpu.async_copy(
                table_hbm.at[idx_v.at[pl.ds(off, CHUNK)]], rows_v, gsem
            ).wait()
            pltpu.sync_copy(rows_v, out_hbm.at[pl.ds(base + off, CHUNK)])
            return carry

        lax.fori_loop(0, n_chunks, body, 0)

    return gather_kernel


def kernel(idx, table):
    B, T = idx.shape
    V, D = table.shape
    N = B * T
    out = _make_gather(N, V, D)(idx.reshape(N).astype(jnp.int32), table)
    return out.reshape(B, T, D)


# speedup vs baseline: 1.0319x; 1.0176x over previous
import functools

import jax
import jax.numpy as jnp
from jax import lax
from jax.experimental import pallas as pl
from jax.experimental.pallas import tpu as pltpu
from jax.experimental.pallas import tpu_sc as plsc

NUM_CORES = 2
NUM_SUBCORES = 16
NUM_WORKERS = NUM_CORES * NUM_SUBCORES
CHUNK = 64


def _make_gather(N, V, D):
    n_per_w = N // NUM_WORKERS
    n_chunks = n_per_w // CHUNK

    mesh = plsc.VectorSubcoreMesh(
        core_axis_name="c", subcore_axis_name="s",
        num_cores=NUM_CORES, num_subcores=NUM_SUBCORES)

    @functools.partial(
        pl.kernel,
        mesh=mesh,
        compiler_params=pltpu.CompilerParams(use_tc_tiling_on_sc=False),
        out_type=jax.ShapeDtypeStruct((N, D), jnp.float32),
        scratch_types=[
            pltpu.VMEM((n_per_w,), jnp.int32),
            pltpu.VMEM((CHUNK, D), jnp.float32),
            pltpu.VMEM((CHUNK, D), jnp.float32),
            pltpu.SemaphoreType.DMA,
            pltpu.SemaphoreType.DMA,
        ],
    )
    def gather_kernel(idx_hbm, table_hbm, out_hbm, idx_v, buf0, buf1, gsem, ssem):
        wid = lax.axis_index("s") * NUM_CORES + lax.axis_index("c")
        base = wid * n_per_w
        bufs = (buf0, buf1)

        pltpu.sync_copy(idx_hbm.at[pl.ds(base, n_per_w)], idx_v)

        def issue_gather(g, buf):
            pltpu.async_copy(
                table_hbm.at[idx_v.at[pl.ds(g * CHUNK, CHUNK)]], buf, gsem)

        def wait_gather(buf):
            pltpu.make_async_copy(
                table_hbm.at[idx_v.at[pl.ds(0, CHUNK)]], buf, gsem).wait()

        def issue_scatter(g, buf):
            pltpu.async_copy(buf, out_hbm.at[pl.ds(base + g * CHUNK, CHUNK)], ssem)

        def wait_scatter(buf):
            pltpu.make_async_copy(buf, out_hbm.at[pl.ds(base, CHUNK)], ssem).wait()

        def step(g, bcur, bnext, refill, swait):
            if swait:
                wait_scatter(bufs[bcur])
            if refill:
                issue_gather(g + 1, bufs[bnext])
            wait_gather(bufs[bcur])
            issue_scatter(g, bufs[bcur])

        issue_gather(0, bufs[0])
        step(0, 0, 1, refill=True, swait=False)

        n_rounds = (n_chunks - 1) // 2
        def round_body(r, carry):
            g = 1 + 2 * r
            step(g, 1, 0, refill=True, swait=True)
            step(g + 1, 0, 1, refill=True, swait=True)
            return carry

        lax.fori_loop(0, n_rounds - 1, round_body, 0)
        g_last = 1 + 2 * (n_rounds - 1)
        step(g_last, 1, 0, refill=True, swait=True)
        step(g_last + 1, 0, 1, refill=False, swait=True)

        wait_scatter(bufs[0])

    return gather_kernel


def kernel(idx, table):
    B, T = idx.shape
    V, D = table.shape
    N = B * T
    out = _make_gather(N, V, D)(idx.reshape(N).astype(jnp.int32), table)
    return out.reshape(B, T, D)


# speedup vs baseline: 1.1449x; 1.1095x over previous
import functools

import jax
import jax.numpy as jnp
from jax import lax
from jax.experimental import pallas as pl
from jax.experimental.pallas import tpu as pltpu
from jax.experimental.pallas import tpu_sc as plsc

NUM_CORES = 2
NUM_SUBCORES = 16
NUM_WORKERS = NUM_CORES * NUM_SUBCORES
CHUNK = 32


def _make_gather(N, V, D):
    n_per_w = N // NUM_WORKERS
    n_chunks = n_per_w // CHUNK

    mesh = plsc.VectorSubcoreMesh(
        core_axis_name="c", subcore_axis_name="s",
        num_cores=NUM_CORES, num_subcores=NUM_SUBCORES)

    @functools.partial(
        pl.kernel,
        mesh=mesh,
        compiler_params=pltpu.CompilerParams(use_tc_tiling_on_sc=False),
        out_type=jax.ShapeDtypeStruct((N, D), jnp.float32),
        scratch_types=[
            pltpu.VMEM((n_per_w,), jnp.int32),
            pltpu.VMEM((CHUNK, D), jnp.float32),
            pltpu.VMEM((CHUNK, D), jnp.float32),
            pltpu.VMEM_SHARED((V, D), jnp.float32),
            pltpu.SemaphoreType.DMA,
            pltpu.SemaphoreType.DMA,
        ],
    )
    def gather_kernel(idx_hbm, table_hbm, out_hbm, idx_v, buf0, buf1, sp_table,
                      gsem, ssem):
        wid = lax.axis_index("s") * NUM_CORES + lax.axis_index("c")
        sid = lax.axis_index("s")
        base = wid * n_per_w
        bufs = (buf0, buf1)

        SLAB = 63
        row0 = jnp.minimum(sid * SLAB, V - SLAB)
        pltpu.sync_copy(table_hbm.at[pl.ds(row0, SLAB)],
                        sp_table.at[pl.ds(row0, SLAB)])
        pltpu.sync_copy(idx_hbm.at[pl.ds(base, n_per_w)], idx_v)
        plsc.subcore_barrier()

        def issue_gather(g, buf):
            pltpu.async_copy(
                sp_table.at[idx_v.at[pl.ds(g * CHUNK, CHUNK)]], buf, gsem)

        def wait_gather(buf):
            pltpu.make_async_copy(
                sp_table.at[idx_v.at[pl.ds(0, CHUNK)]], buf, gsem).wait()

        def issue_scatter(g, buf):
            pltpu.async_copy(buf, out_hbm.at[pl.ds(base + g * CHUNK, CHUNK)], ssem)

        def wait_scatter(buf):
            pltpu.make_async_copy(buf, out_hbm.at[pl.ds(base, CHUNK)], ssem).wait()

        def step(g, bcur, bnext, refill, swait):
            if swait:
                wait_scatter(bufs[bcur])
            if refill:
                issue_gather(g + 1, bufs[bnext])
            wait_gather(bufs[bcur])
            issue_scatter(g, bufs[bcur])

        issue_gather(0, bufs[0])
        step(0, 0, 1, refill=True, swait=False)

        n_rounds = (n_chunks - 2) // 2
        def round_body(r, carry):
            g = 1 + 2 * r
            step(g, 1, 0, refill=True, swait=True)
            step(g + 1, 0, 1, refill=True, swait=True)
            return carry

        lax.fori_loop(0, n_rounds, round_body, 0)
        step(n_chunks - 1, 1, 0, refill=False, swait=True)

        wait_scatter(bufs[1])

    return gather_kernel


def kernel(idx, table):
    B, T = idx.shape
    V, D = table.shape
    N = B * T
    out = _make_gather(N, V, D)(idx.reshape(N).astype(jnp.int32), table)
    return out.reshape(B, T, D)
